# baseline (speedup 1.0000x reference)
import functools

import jax
import jax.numpy as jnp
from jax import lax
from jax.experimental import pallas as pl
from jax.experimental.pallas import tpu as pltpu
from jax.experimental.pallas import tpu_sc as plsc

Q = 1024
D = 64
N = 100000
L = 128
K = 50

NCHUNK = 8
W1 = 12544
N_PAD = NCHUNK * W1
W2 = 784
W3 = 49
SBLK = 1792
NSEL = 56
FINE = NSEL * NCHUNK
NEG = -1e30
QH = Q // 4


def _rne_bf16_tc(x):
    u = lax.bitcast_convert_type(x, jnp.int32)
    r = (u + 0x7FFF + ((u >> 16) & 1)) & jnp.int32(-65536)
    return lax.bitcast_convert_type(r, jnp.float32)


def _make_tc_body(with_mn):
    def body(qf_ref, mem_ref, qn_ref, *rest):
        mn_ref, g1_ref = rest if with_mn else (None, rest[0])
        s = pl.program_id(0)
        c = pl.program_id(1)

        q = qf_ref[...]
        qn = q / jnp.maximum(
            jnp.sqrt(jnp.sum(q * q, axis=1, keepdims=True)), 1e-12)

        @pl.when(jnp.logical_and(s == 0, c == 0))
        def _():
            qn_ref[...] = jnp.concatenate(
                [_rne_bf16_tc(qn), jnp.zeros_like(qn)], axis=1)

        m = mem_ref[...]
        mn = m / jnp.maximum(
            jnp.sqrt(jnp.sum(m * m, axis=1, keepdims=True)), 1e-12)
        if with_mn:
            mn_ref[...] = jnp.concatenate(
                [_rne_bf16_tc(mn), jnp.zeros_like(mn)], axis=1)

        sims = lax.dot_general(qn, mn, (((1,), (1,)), ((), ())),
                               preferred_element_type=jnp.float32)
        last = jnp.logical_and(c == NCHUNK - 1, s == W1 // SBLK - 1)

        @pl.when(last)
        def _():
            base = c * W1 + s * SBLK
            cols = base + lax.broadcasted_iota(jnp.int32, (QH, SBLK), 1)
            masked = jnp.where(cols >= N, NEG, sims)
            g1_ref[...] = jnp.maximum(g1_ref[...], masked)

        @pl.when(jnp.logical_and(jnp.logical_not(last), c == 0))
        def _():
            g1_ref[...] = sims

        @pl.when(jnp.logical_and(jnp.logical_not(last), c != 0))
        def _():
            g1_ref[...] = jnp.maximum(g1_ref[...], sims)

    return body


def _tc_stage(qf_half, mem_pad, with_mn):
    out_specs = [
        pl.BlockSpec((QH, 2 * D), lambda s, c: (0, 0)),
        pl.BlockSpec((SBLK, 2 * D), lambda s, c: (c * (W1 // SBLK) + s, 0)),
        pl.BlockSpec((QH, SBLK), lambda s, c: (0, s)),
    ]
    out_shape = [
        jax.ShapeDtypeStruct((QH, 2 * D), jnp.float32),
        jax.ShapeDtypeStruct((N_PAD, 2 * D), jnp.float32),
        jax.ShapeDtypeStruct((QH, W1), jnp.float32),
    ]
    if not with_mn:
        del out_specs[1], out_shape[1]
    return pl.pallas_call(
        _make_tc_body(with_mn),
        grid=(W1 // SBLK, NCHUNK),
        in_specs=[
            pl.BlockSpec((QH, D), lambda s, c: (0, 0)),
            pl.BlockSpec((SBLK, D), lambda s, c: (c * (W1 // SBLK) + s, 0)),
        ],
        out_specs=out_specs,
        out_shape=out_shape,
    )(qf_half, mem_pad)


def _shuf(v, s):
    return jnp.take_along_axis(v, lax.iota(jnp.int32, 16) ^ s, axis=0)


def _bmax(v):
    for s in (8, 4, 2, 1):
        v = jnp.maximum(v, _shuf(v, s))
    return v


def _bmin_i(v):
    for s in (8, 4, 2, 1):
        v = jnp.minimum(v, _shuf(v, s))
    return v


def _bsum(v):
    for s in (8, 4, 2, 1):
        v = v + _shuf(v, s)
    return v


def _amax_sel(vals, payload):
    m = _bmax(vals)
    cand = jnp.where(vals == m, payload, jnp.int32(0x7FFFFFFF))
    return m, _bmin_i(cand)


def _splat_i(x):
    return lax.iota(jnp.int32, 16) * 0 + x


def _splat_f(x):
    return jnp.zeros((16,), jnp.float32) + x


def _sc_body(qn_hbm, mn_hbm, g1_hbm, lab_hbm, out_hbm,
             g1d, g2, g3, groups, fidx, fidxh, rows, fvals, ga,
             wbuf, labs, qd, outd, sem, seml, sempg, sempq, semo):
    cid = lax.axis_index("c")
    sid = lax.axis_index("s")
    wid = sid * 2 + cid
    iota = lax.iota(jnp.int32, 16)
    lane0 = iota == 0
    base_q = wid * (QH // 32)

    pltpu.async_copy(g1_hbm.at[base_q], g1d.at[0], sempg)
    pltpu.async_copy(qn_hbm.at[base_q], qd.at[0], sempq)

    def per_query(qi, carry):
        qrow_idx = base_q + qi
        p = qi & 1

        pltpu.make_async_copy(g1_hbm.at[qrow_idx], g1d.at[p], sempg).wait()
        pltpu.make_async_copy(qn_hbm.at[qrow_idx], qd.at[p], sempq).wait()

        @pl.when(qi >= 2)
        def _():
            pltpu.make_async_copy(outd.at[p], out_hbm.at[qrow_idx],
                                  semo).wait()

        @pl.when(qi < QH // 32 - 1)
        def _():
            pltpu.async_copy(g1_hbm.at[qrow_idx + 1], g1d.at[1 - p], sempg)
            pltpu.async_copy(qn_hbm.at[qrow_idx + 1], qd.at[1 - p], sempq)

        def g2_body(jb, _):
            off = jb * 16
            acc = g1d[p, pl.ds(off, 16)]
            for i in range(1, 16):
                acc = jnp.maximum(acc, g1d[p, pl.ds(i * W2 + off, 16)])
            g2[pl.ds(off, 16)] = acc
            return 0

        lax.fori_loop(0, W3, g2_body, 0, unroll=False)

        for jb in range(4):
            jv = iota + jb * 16
            valid = jv < W3
            jvs = jnp.where(valid, jv, 0)
            acc = _splat_f(NEG)
            for i in range(16):
                v = plsc.load_gather(g2, [jvs + i * W3])
                acc = jnp.maximum(acc, jnp.where(valid, v, NEG))
            g3[pl.ds(jb * 16, 16)] = acc

        def ext_body(t, _):
            mv = g3[pl.ds(0, 16)]
            iv = iota
            for b in range(1, 4):
                v = g3[pl.ds(b * 16, 16)]
                gt = v > mv
                mv = jnp.where(gt, v, mv)
                iv = jnp.where(gt, iota + b * 16, iv)
            _, j3 = _amax_sel(mv, iv)

            v2 = plsc.load_gather(g2, [iota * W3 + j3])
            _, i2 = _amax_sel(v2, iota)
            j2 = i2 * W3 + j3

            v1 = plsc.load_gather(g1d, [_splat_i(p), iota * W2 + j2])
            _, i1 = _amax_sel(v1, iota)
            grp = i1 * W2 + j2

            plsc.store_scatter(groups, [_splat_i(t)], grp, mask=lane0)
            plsc.store_scatter(g1d, [_splat_i(p), grp], _splat_f(NEG),
                               mask=lane0)

            ng2 = _bmax(jnp.where(iota == i1, NEG, v1))
            plsc.store_scatter(g2, [j2], ng2, mask=lane0)
            ng3 = _bmax(jnp.where(iota == i2, ng2, v2))
            plsc.store_scatter(g3, [j3], ng3, mask=lane0)

            @pl.when((t & 15) == 15)
            def _():
                j = t >> 4
                for k in range(8):
                    fl = k * 16
                    gidx = j * 16 + 2 * k + (iota >> 3)
                    gv = plsc.load_gather(groups, [gidx])
                    fidx[j, pl.ds(fl, 16)] = gv + (iota & 7) * W1
                pltpu.async_copy(mn_hbm.at[fidx.at[j]],
                                 rows.at[pl.ds(j * 128, 128)], sem)
            return 0

        lax.fori_loop(0, NSEL, ext_body, 0, unroll=False)

        for k in range(4):
            fl = k * 16
            gidx = 48 + 2 * k + (iota >> 3)
            gv = plsc.load_gather(groups, [gidx])
            vals = gv + (iota & 7) * W1
            fidxh[pl.ds(fl, 16)] = vals
            fidx[3, pl.ds(fl, 16)] = vals
        pltpu.async_copy(mn_hbm.at[fidxh], rows.at[pl.ds(384, 64)], sem)

        q0 = qd[p, pl.ds(0, 16)]
        q1 = qd[p, pl.ds(16, 16)]
        q2 = qd[p, pl.ds(32, 16)]
        q3 = qd[p, pl.ds(48, 16)]

        def dot2_body(i, _):
            ra = i * 2
            aa = (rows[ra, pl.ds(0, 16)] * q0 + rows[ra, pl.ds(16, 16)] * q1
                  + rows[ra, pl.ds(32, 16)] * q2
                  + rows[ra, pl.ds(48, 16)] * q3)
            rb = ra + 1
            ab = (rows[rb, pl.ds(0, 16)] * q0 + rows[rb, pl.ds(16, 16)] * q1
                  + rows[rb, pl.ds(32, 16)] * q2
                  + rows[rb, pl.ds(48, 16)] * q3)
            ta = aa + _shuf(aa, 8)
            tb = ab + _shuf(ab, 8)
            m = jnp.where(iota < 8, ta, _shuf(tb, 8))
            for s in (4, 2, 1):
                m = m + _shuf(m, s)
            plsc.store_scatter(fvals, [_splat_i(ra) + (iota >> 3)], m,
                               mask=(iota & 7) == 0)
            return 0

        for j in range(3):
            pltpu.make_async_copy(mn_hbm.at[fidx.at[j]],
                                  rows.at[pl.ds(j * 128, 128)], sem).wait()
            lax.fori_loop(j * 64, (j + 1) * 64, dot2_body, 0, unroll=False)
        pltpu.make_async_copy(mn_hbm.at[fidxh], rows.at[pl.ds(384, 64)],
                              sem).wait()
        lax.fori_loop(192, 224, dot2_body, 0, unroll=False)

        for rb in range(FINE // 16):
            flat = rb * 16
            iv = fidx[flat // 128, pl.ds(flat % 128, 16)]
            fv = fvals[pl.ds(flat, 16)]
            fvals[pl.ds(flat, 16)] = jnp.where(iv >= N, NEG, fv)

        for b in range(FINE // 16, 32):
            fvals[pl.ds(b * 16, 16)] = _splat_f(NEG)

        for jb in range(2):
            off = jb * 16
            acc = fvals[pl.ds(off, 16)]
            for i in range(1, 16):
                acc = jnp.maximum(acc, fvals[pl.ds(i * 32 + off, 16)])
            ga[pl.ds(off, 16)] = acc

        for b in range(4):
            wbuf[pl.ds(b * 16, 16)] = jnp.zeros((16,), jnp.float32)

        def top_body(t, _):
            mv = ga[pl.ds(0, 16)]
            iv = iota
            v = ga[pl.ds(16, 16)]
            gt = v > mv
            mv = jnp.where(gt, v, mv)
            iv = jnp.where(gt, iota + 16, iv)
            m, ja = _amax_sel(mv, iv)

            vc = plsc.load_gather(fvals, [iota * 32 + ja])
            _, if_ = _amax_sel(vc, iota)
            r = if_ * 32 + ja

            nv = plsc.load_gather(fidx, [r >> 7, r & 127])
            pltpu.async_copy(lab_hbm.at[nv[0]], labs.at[t], seml)

            plsc.store_scatter(wbuf, [_splat_i(t)], m, mask=lane0)
            plsc.store_scatter(fvals, [r], _splat_f(NEG), mask=lane0)
            nga = _bmax(jnp.where(iota == if_, NEG, vc))
            plsc.store_scatter(ga, [ja], nga, mask=lane0)
            return 0

        lax.fori_loop(0, K, top_body, 0, unroll=False)

        def drain_body(t, _):
            pltpu.make_async_copy(lab_hbm.at[0], labs.at[t], seml).wait()
            return 0

        lax.fori_loop(0, K, drain_body, 0, unroll=False)

        def vote_body(t, accs):
            wv = plsc.load_gather(wbuf, [_splat_i(t)])
            return tuple(accs[i] + wv * labs[t, pl.ds(i * 16, 16)]
                         for i in range(8))

        accs = lax.fori_loop(0, K, vote_body,
                             tuple(jnp.zeros((16,), jnp.float32)
                                   for _ in range(8)), unroll=False)

        wacc = wbuf[pl.ds(0, 16)]
        for b in range(1, 4):
            wacc = wacc + wbuf[pl.ds(b * 16, 16)]
        inv = jnp.ones((16,), jnp.float32) / (_bsum(wacc) + 1e-8)
        for i in range(8):
            score = accs[i] * inv
            outd[p, pl.ds(i * 16, 16)] = (score >= 0.3).astype(jnp.int32)
        pltpu.async_copy(outd.at[p], out_hbm.at[qrow_idx], semo)
        return carry

    lax.fori_loop(0, QH // 32, per_query, 0, unroll=False)

    last = base_q + QH // 32 - 1
    pltpu.make_async_copy(outd.at[0], out_hbm.at[last], semo).wait()
    pltpu.make_async_copy(outd.at[1], out_hbm.at[last], semo).wait()


def _sc_stage(qn, mn, g1, labels):
    mesh = plsc.VectorSubcoreMesh(core_axis_name="c", subcore_axis_name="s")
    f = functools.partial(
        pl.kernel,
        mesh=mesh,
        compiler_params=pltpu.CompilerParams(needs_layout_passes=False),
        out_type=jax.ShapeDtypeStruct((QH, L), jnp.int32),
        scratch_types=[
            pltpu.VMEM((2, W1), jnp.float32),
            pltpu.VMEM((W2,), jnp.float32),
            pltpu.VMEM((64,), jnp.float32),
            pltpu.VMEM((64,), jnp.int32),
            pltpu.VMEM((4, 128), jnp.int32),
            pltpu.VMEM((64,), jnp.int32),
            pltpu.VMEM((FINE, 2 * D), jnp.float32),
            pltpu.VMEM((512,), jnp.float32),
            pltpu.VMEM((32,), jnp.float32),
            pltpu.VMEM((64,), jnp.float32),
            pltpu.VMEM((64, L), jnp.float32),
            pltpu.VMEM((2, 2 * D), jnp.float32),
            pltpu.VMEM((2, L), jnp.int32),
            pltpu.SemaphoreType.DMA,
            pltpu.SemaphoreType.DMA,
            pltpu.SemaphoreType.DMA,
            pltpu.SemaphoreType.DMA,
            pltpu.SemaphoreType.DMA,
        ],
    )(_sc_body)
    return f(qn, mn, g1, labels)


def kernel(query_features, memory_features, memory_labels):
    mem_pad = jnp.pad(memory_features, ((0, N_PAD - N), (0, 0)))
    parts = []
    mn = None
    for i in range(Q // QH):
        qf_i = query_features[i * QH:(i + 1) * QH]
        if mn is None:
            qn_i, mn, g1_i = _tc_stage(qf_i, mem_pad, True)
        else:
            qn_i, g1_i = _tc_stage(qf_i, mem_pad, False)
        parts.append((qn_i, g1_i))
    preds = [_sc_stage(qn_i, mn, g1_i, memory_labels)
             for qn_i, g1_i in parts]
    return jnp.concatenate(preds, axis=0)


# speedup vs baseline: 1.0069x; 1.0069x over previous
import functools

import jax
import jax.numpy as jnp
from jax import lax
from jax.experimental import pallas as pl
from jax.experimental.pallas import tpu as pltpu
from jax.experimental.pallas import tpu_sc as plsc

Q = 1024
D = 64
N = 100000
L = 128
K = 50

NCHUNK = 8
W1 = 12544
N_PAD = NCHUNK * W1
W2 = 784
W3 = 49
SBLK = 1792
NSEL = 56
FINE = NSEL * NCHUNK
NEG = -1e30
QH = Q // 2


def _rne_bf16_tc(x):
    u = lax.bitcast_convert_type(x, jnp.int32)
    r = (u + 0x7FFF + ((u >> 16) & 1)) & jnp.int32(-65536)
    return lax.bitcast_convert_type(r, jnp.float32)


def _make_tc_body(with_mn):
    def body(qf_ref, mem_ref, qn_ref, *rest):
        mn_ref, g1_ref = rest if with_mn else (None, rest[0])
        s = pl.program_id(0)
        c = pl.program_id(1)

        q = qf_ref[...]
        qn = q / jnp.maximum(
            jnp.sqrt(jnp.sum(q * q, axis=1, keepdims=True)), 1e-12)

        @pl.when(jnp.logical_and(s == 0, c == 0))
        def _():
            qn_ref[...] = jnp.concatenate(
                [_rne_bf16_tc(qn), jnp.zeros_like(qn)], axis=1)

        m = mem_ref[...]
        mn = m / jnp.maximum(
            jnp.sqrt(jnp.sum(m * m, axis=1, keepdims=True)), 1e-12)
        if with_mn:
            mn_ref[...] = jnp.concatenate(
                [_rne_bf16_tc(mn), jnp.zeros_like(mn)], axis=1)

        sims = lax.dot_general(qn, mn, (((1,), (1,)), ((), ())),
                               preferred_element_type=jnp.float32)
        last = jnp.logical_and(c == NCHUNK - 1, s == W1 // SBLK - 1)

        @pl.when(last)
        def _():
            base = c * W1 + s * SBLK
            cols = base + lax.broadcasted_iota(jnp.int32, (QH, SBLK), 1)
            masked = jnp.where(cols >= N, NEG, sims)
            g1_ref[...] = jnp.maximum(g1_ref[...], masked)

        @pl.when(jnp.logical_and(jnp.logical_not(last), c == 0))
        def _():
            g1_ref[...] = sims

        @pl.when(jnp.logical_and(jnp.logical_not(last), c != 0))
        def _():
            g1_ref[...] = jnp.maximum(g1_ref[...], sims)

    return body


def _tc_stage(qf_half, mem_pad, with_mn):
    out_specs = [
        pl.BlockSpec((QH, 2 * D), lambda s, c: (0, 0)),
        pl.BlockSpec((SBLK, 2 * D), lambda s, c: (c * (W1 // SBLK) + s, 0)),
        pl.BlockSpec((QH, SBLK), lambda s, c: (0, s)),
    ]
    out_shape = [
        jax.ShapeDtypeStruct((QH, 2 * D), jnp.float32),
        jax.ShapeDtypeStruct((N_PAD, 2 * D), jnp.float32),
        jax.ShapeDtypeStruct((QH, W1), jnp.float32),
    ]
    if not with_mn:
        del out_specs[1], out_shape[1]
    return pl.pallas_call(
        _make_tc_body(with_mn),
        grid=(W1 // SBLK, NCHUNK),
        in_specs=[
            pl.BlockSpec((QH, D), lambda s, c: (0, 0)),
            pl.BlockSpec((SBLK, D), lambda s, c: (c * (W1 // SBLK) + s, 0)),
        ],
        out_specs=out_specs,
        out_shape=out_shape,
    )(qf_half, mem_pad)


def _shuf(v, s):
    return jnp.take_along_axis(v, lax.iota(jnp.int32, 16) ^ s, axis=0)


def _bmax(v):
    for s in (8, 4, 2, 1):
        v = jnp.maximum(v, _shuf(v, s))
    return v


def _bmin_i(v):
    for s in (8, 4, 2, 1):
        v = jnp.minimum(v, _shuf(v, s))
    return v


def _bsum(v):
    for s in (8, 4, 2, 1):
        v = v + _shuf(v, s)
    return v


def _amax_sel(vals, payload):
    m = _bmax(vals)
    cand = jnp.where(vals == m, payload, jnp.int32(0x7FFFFFFF))
    return m, _bmin_i(cand)


def _splat_i(x):
    return lax.iota(jnp.int32, 16) * 0 + x


def _splat_f(x):
    return jnp.zeros((16,), jnp.float32) + x


def _sc_body(qn_hbm, mn_hbm, g1_hbm, lab_hbm, out_hbm,
             g1d, g2, g3, groups, fidx, fidxh, rows, fvals, ga,
             wbuf, labs, qd, outd, sem, seml, sempg, sempq, semo):
    cid = lax.axis_index("c")
    sid = lax.axis_index("s")
    wid = sid * 2 + cid
    iota = lax.iota(jnp.int32, 16)
    lane0 = iota == 0
    base_q = wid * (QH // 32)

    pltpu.async_copy(g1_hbm.at[base_q], g1d.at[0], sempg)
    pltpu.async_copy(qn_hbm.at[base_q], qd.at[0], sempq)

    def per_query(qi, carry):
        qrow_idx = base_q + qi
        p = qi & 1

        pltpu.make_async_copy(g1_hbm.at[qrow_idx], g1d.at[p], sempg).wait()
        pltpu.make_async_copy(qn_hbm.at[qrow_idx], qd.at[p], sempq).wait()

        @pl.when(qi >= 2)
        def _():
            pltpu.make_async_copy(outd.at[p], out_hbm.at[qrow_idx],
                                  semo).wait()

        @pl.when(qi < QH // 32 - 1)
        def _():
            pltpu.async_copy(g1_hbm.at[qrow_idx + 1], g1d.at[1 - p], sempg)
            pltpu.async_copy(qn_hbm.at[qrow_idx + 1], qd.at[1 - p], sempq)

        def g2_body(jb, _):
            off = jb * 16
            acc = g1d[p, pl.ds(off, 16)]
            for i in range(1, 16):
                acc = jnp.maximum(acc, g1d[p, pl.ds(i * W2 + off, 16)])
            g2[pl.ds(off, 16)] = acc
            return 0

        lax.fori_loop(0, W3, g2_body, 0, unroll=False)

        for jb in range(4):
            jv = iota + jb * 16
            valid = jv < W3
            jvs = jnp.where(valid, jv, 0)
            acc = _splat_f(NEG)
            for i in range(16):
                v = plsc.load_gather(g2, [jvs + i * W3])
                acc = jnp.maximum(acc, jnp.where(valid, v, NEG))
            g3[pl.ds(jb * 16, 16)] = acc

        def ext_body(t, _):
            mv = g3[pl.ds(0, 16)]
            iv = iota
            for b in range(1, 4):
                v = g3[pl.ds(b * 16, 16)]
                gt = v > mv
                mv = jnp.where(gt, v, mv)
                iv = jnp.where(gt, iota + b * 16, iv)
            _, j3 = _amax_sel(mv, iv)

            v2 = plsc.load_gather(g2, [iota * W3 + j3])
            _, i2 = _amax_sel(v2, iota)
            j2 = i2 * W3 + j3

            v1 = plsc.load_gather(g1d, [_splat_i(p), iota * W2 + j2])
            _, i1 = _amax_sel(v1, iota)
            grp = i1 * W2 + j2

            plsc.store_scatter(groups, [_splat_i(t)], grp, mask=lane0)
            plsc.store_scatter(g1d, [_splat_i(p), grp], _splat_f(NEG),
                               mask=lane0)

            ng2 = _bmax(jnp.where(iota == i1, NEG, v1))
            plsc.store_scatter(g2, [j2], ng2, mask=lane0)
            ng3 = _bmax(jnp.where(iota == i2, ng2, v2))
            plsc.store_scatter(g3, [j3], ng3, mask=lane0)

            @pl.when((t & 15) == 15)
            def _():
                j = t >> 4
                for k in range(8):
                    fl = k * 16
                    gidx = j * 16 + 2 * k + (iota >> 3)
                    gv = plsc.load_gather(groups, [gidx])
                    fidx[j, pl.ds(fl, 16)] = gv + (iota & 7) * W1
                pltpu.async_copy(mn_hbm.at[fidx.at[j]],
                                 rows.at[pl.ds(j * 128, 128)], sem)
            return 0

        lax.fori_loop(0, NSEL, ext_body, 0, unroll=False)

        for k in range(4):
            fl = k * 16
            gidx = 48 + 2 * k + (iota >> 3)
            gv = plsc.load_gather(groups, [gidx])
            vals = gv + (iota & 7) * W1
            fidxh[pl.ds(fl, 16)] = vals
            fidx[3, pl.ds(fl, 16)] = vals
        pltpu.async_copy(mn_hbm.at[fidxh], rows.at[pl.ds(384, 64)], sem)

        q0 = qd[p, pl.ds(0, 16)]
        q1 = qd[p, pl.ds(16, 16)]
        q2 = qd[p, pl.ds(32, 16)]
        q3 = qd[p, pl.ds(48, 16)]

        def dot2_body(i, _):
            ra = i * 2
            aa = (rows[ra, pl.ds(0, 16)] * q0 + rows[ra, pl.ds(16, 16)] * q1
                  + rows[ra, pl.ds(32, 16)] * q2
                  + rows[ra, pl.ds(48, 16)] * q3)
            rb = ra + 1
            ab = (rows[rb, pl.ds(0, 16)] * q0 + rows[rb, pl.ds(16, 16)] * q1
                  + rows[rb, pl.ds(32, 16)] * q2
                  + rows[rb, pl.ds(48, 16)] * q3)
            ta = aa + _shuf(aa, 8)
            tb = ab + _shuf(ab, 8)
            m = jnp.where(iota < 8, ta, _shuf(tb, 8))
            for s in (4, 2, 1):
                m = m + _shuf(m, s)
            plsc.store_scatter(fvals, [_splat_i(ra) + (iota >> 3)], m,
                               mask=(iota & 7) == 0)
            return 0

        for j in range(3):
            pltpu.make_async_copy(mn_hbm.at[fidx.at[j]],
                                  rows.at[pl.ds(j * 128, 128)], sem).wait()
            lax.fori_loop(j * 64, (j + 1) * 64, dot2_body, 0, unroll=False)
        pltpu.make_async_copy(mn_hbm.at[fidxh], rows.at[pl.ds(384, 64)],
                              sem).wait()
        lax.fori_loop(192, 224, dot2_body, 0, unroll=False)

        for rb in range(FINE // 16):
            flat = rb * 16
            iv = fidx[flat // 128, pl.ds(flat % 128, 16)]
            fv = fvals[pl.ds(flat, 16)]
            fvals[pl.ds(flat, 16)] = jnp.where(iv >= N, NEG, fv)

        for b in range(FINE // 16, 32):
            fvals[pl.ds(b * 16, 16)] = _splat_f(NEG)

        for jb in range(2):
            off = jb * 16
            acc = fvals[pl.ds(off, 16)]
            for i in range(1, 16):
                acc = jnp.maximum(acc, fvals[pl.ds(i * 32 + off, 16)])
            ga[pl.ds(off, 16)] = acc

        for b in range(4):
            wbuf[pl.ds(b * 16, 16)] = jnp.zeros((16,), jnp.float32)

        def top_body(t, _):
            mv = ga[pl.ds(0, 16)]
            iv = iota
            v = ga[pl.ds(16, 16)]
            gt = v > mv
            mv = jnp.where(gt, v, mv)
            iv = jnp.where(gt, iota + 16, iv)
            m, ja = _amax_sel(mv, iv)

            vc = plsc.load_gather(fvals, [iota * 32 + ja])
            _, if_ = _amax_sel(vc, iota)
            r = if_ * 32 + ja

            nv = plsc.load_gather(fidx, [r >> 7, r & 127])
            pltpu.async_copy(lab_hbm.at[nv[0]], labs.at[t], seml)

            plsc.store_scatter(wbuf, [_splat_i(t)], m, mask=lane0)
            plsc.store_scatter(fvals, [r], _splat_f(NEG), mask=lane0)
            nga = _bmax(jnp.where(iota == if_, NEG, vc))
            plsc.store_scatter(ga, [ja], nga, mask=lane0)
            return 0

        lax.fori_loop(0, K, top_body, 0, unroll=False)

        def drain_body(t, _):
            pltpu.make_async_copy(lab_hbm.at[0], labs.at[t], seml).wait()
            return 0

        lax.fori_loop(0, K, drain_body, 0, unroll=False)

        def vote_body(t, accs):
            wv = plsc.load_gather(wbuf, [_splat_i(t)])
            return tuple(accs[i] + wv * labs[t, pl.ds(i * 16, 16)]
                         for i in range(8))

        accs = lax.fori_loop(0, K, vote_body,
                             tuple(jnp.zeros((16,), jnp.float32)
                                   for _ in range(8)), unroll=False)

        wacc = wbuf[pl.ds(0, 16)]
        for b in range(1, 4):
            wacc = wacc + wbuf[pl.ds(b * 16, 16)]
        inv = jnp.ones((16,), jnp.float32) / (_bsum(wacc) + 1e-8)
        for i in range(8):
            score = accs[i] * inv
            outd[p, pl.ds(i * 16, 16)] = (score >= 0.3).astype(jnp.int32)
        pltpu.async_copy(outd.at[p], out_hbm.at[qrow_idx], semo)
        return carry

    lax.fori_loop(0, QH // 32, per_query, 0, unroll=False)

    last = base_q + QH // 32 - 1
    pltpu.make_async_copy(outd.at[0], out_hbm.at[last], semo).wait()
    pltpu.make_async_copy(outd.at[1], out_hbm.at[last], semo).wait()


def _sc_stage(qn, mn, g1, labels):
    mesh = plsc.VectorSubcoreMesh(core_axis_name="c", subcore_axis_name="s")
    f = functools.partial(
        pl.kernel,
        mesh=mesh,
        compiler_params=pltpu.CompilerParams(needs_layout_passes=False),
        out_type=jax.ShapeDtypeStruct((QH, L), jnp.int32),
        scratch_types=[
            pltpu.VMEM((2, W1), jnp.float32),
            pltpu.VMEM((W2,), jnp.float32),
            pltpu.VMEM((64,), jnp.float32),
            pltpu.VMEM((64,), jnp.int32),
            pltpu.VMEM((4, 128), jnp.int32),
            pltpu.VMEM((64,), jnp.int32),
            pltpu.VMEM((FINE, 2 * D), jnp.float32),
            pltpu.VMEM((512,), jnp.float32),
            pltpu.VMEM((32,), jnp.float32),
            pltpu.VMEM((64,), jnp.float32),
            pltpu.VMEM((64, L), jnp.float32),
            pltpu.VMEM((2, 2 * D), jnp.float32),
            pltpu.VMEM((2, L), jnp.int32),
            pltpu.SemaphoreType.DMA,
            pltpu.SemaphoreType.DMA,
            pltpu.SemaphoreType.DMA,
            pltpu.SemaphoreType.DMA,
            pltpu.SemaphoreType.DMA,
        ],
    )(_sc_body)
    return f(qn, mn, g1, labels)


def kernel(query_features, memory_features, memory_labels):
    mem_pad = jnp.pad(memory_features, ((0, N_PAD - N), (0, 0)))
    parts = []
    mn = None
    for i in range(Q // QH):
        qf_i = query_features[i * QH:(i + 1) * QH]
        if mn is None:
            qn_i, mn, g1_i = _tc_stage(qf_i, mem_pad, True)
        else:
            qn_i, g1_i = _tc_stage(qf_i, mem_pad, False)
        parts.append((qn_i, g1_i))
    preds = [_sc_stage(qn_i, mn, g1_i, memory_labels)
             for qn_i, g1_i in parts]
    return jnp.concatenate(preds, axis=0)
